# Initial kernel scaffold; baseline (speedup 1.0000x reference)
import jax
import jax.numpy as jnp
from jax.experimental import pallas as pl


def kernel(user_features, user_ad_history, target_ad_id, emb_table, dense_table, fc1_w, fc1_b, fc2_w, fc2_b, fc3_w, fc3_b):
    raise NotImplementedError("write your pallas kernel here")


# speedup vs baseline: 2.1527x; 2.1527x over previous
import functools

import jax
import jax.numpy as jnp
from jax import lax
from jax.experimental import pallas as pl
from jax.experimental.pallas import tpu as pltpu
from jax.experimental.pallas import tpu_sc as plsc

B = 1024
L = 200
EM = 160
DD = 8
NC = 2
NS = 16
NW = NC * NS
BPW = B // NW
NCH = 5
CH = L // NCH
EV = EM // 16

_mesh = plsc.VectorSubcoreMesh(core_axis_name="c", subcore_axis_name="s")


@functools.partial(
    pl.kernel,
    out_type=[
        jax.ShapeDtypeStruct((B, EM), jnp.float32),
        jax.ShapeDtypeStruct((B, 16), jnp.float32),
        jax.ShapeDtypeStruct((B, EM), jnp.float32),
        jax.ShapeDtypeStruct((B, DD), jnp.float32),
    ],
    mesh=_mesh,
    compiler_params=pltpu.CompilerParams(use_tc_tiling_on_sc=False,
                                         needs_layout_passes=False),
    scratch_types=[
        pltpu.VMEM((BPW * L,), jnp.int32),
        pltpu.VMEM((BPW,), jnp.int32),
        pltpu.VMEM((NCH, CH, EM), jnp.float32),
        pltpu.VMEM((NCH, CH, DD), jnp.float32),
        pltpu.VMEM((BPW, EM), jnp.float32),
        pltpu.VMEM((BPW, 16), jnp.float32),
        pltpu.VMEM((BPW, EM), jnp.float32),
        pltpu.VMEM((BPW, DD), jnp.float32),
        pltpu.VMEM((32,), jnp.float32),
        pltpu.SemaphoreType.DMA,
    ],
)
def _sc_bag_sums(hist_hbm, tgt_hbm, emb_hbm, dense_hbm,
                 us_hbm, ud_hbm, ts_hbm, td_hbm,
                 idx_v, tgt_v, erows_v, drows_v, us_v, ud_v, trow_v, tdrow_v,
                 mask_v, sem):
    wid = lax.axis_index("s") * NC + lax.axis_index("c")
    base = wid * BPW

    pltpu.sync_copy(hist_hbm.at[pl.ds(base * L, BPW * L)], idx_v)
    pltpu.sync_copy(tgt_hbm.at[pl.ds(base, BPW)], tgt_v)

    pltpu.async_copy(emb_hbm.at[tgt_v], trow_v, sem).wait()
    pltpu.async_copy(dense_hbm.at[tgt_v], tdrow_v, sem).wait()

    lanes = lax.iota(jnp.int32, 16)
    row_off = jnp.where(lanes >= 8, 1, 0)
    col_idx = lanes & 7

    def bag_body(b, _):
        cps = []
        for c in range(NCH):
            sl = idx_v.at[pl.ds(b * L + c * CH, CH)]
            cps.append(pltpu.async_copy(emb_hbm.at[sl], erows_v.at[c], sem))
            cps.append(pltpu.async_copy(dense_hbm.at[sl], drows_v.at[c], sem))
        for cp in cps:
            cp.wait()

        accs = tuple(jnp.zeros((16,), jnp.float32) for _ in range(EV))
        for c in range(NCH):
            def erow_body(r, a, c=c):
                return tuple(x + erows_v[c, r, pl.ds(d * 16, 16)]
                             for d, x in enumerate(a))
            accs = lax.fori_loop(0, CH, erow_body, accs)
        for d, a in enumerate(accs):
            us_v[b, pl.ds(d * 16, 16)] = a

        dacc = jnp.zeros((16,), jnp.float32)
        for c in range(NCH):
            cidx = jnp.zeros((16,), jnp.int32) + c

            def drow_body(i, acc, cidx=cidx):
                ridx = 2 * i + row_off
                return acc + plsc.load_gather(drows_v, [cidx, ridx, col_idx])
            dacc = lax.fori_loop(0, CH // 2, drow_body, dacc)
        mask_v[pl.ds(0, 16)] = dacc
        hi = plsc.load_gather(mask_v, [lanes ^ 8])
        ud_v[b, :] = dacc + hi
        return 0

    lax.fori_loop(0, BPW, bag_body, 0)

    for g in range(BPW // 16):
        tv = tgt_v[pl.ds(g * 16, 16)]
        mask_v[pl.ds(g * 16, 16)] = jnp.where(tv != 0, 1.0, 0.0)

    def mask_body(j, _):
        bc = plsc.load_gather(mask_v, [jnp.zeros((16,), jnp.int32) + j])
        for d in range(EV):
            trow_v[j, pl.ds(d * 16, 16)] = trow_v[j, pl.ds(d * 16, 16)] * bc
        return 0

    lax.fori_loop(0, BPW, mask_body, 0)

    pltpu.sync_copy(us_v, us_hbm.at[pl.ds(base, BPW), :])
    pltpu.sync_copy(ud_v, ud_hbm.at[pl.ds(base, BPW), :])
    pltpu.sync_copy(trow_v, ts_hbm.at[pl.ds(base, BPW), :])
    pltpu.sync_copy(tdrow_v, td_hbm.at[pl.ds(base, BPW), :])


def _mlp_body(x_ref, w1_ref, b1_ref, w2_ref, b2_ref, w3_ref, b3_ref, o_ref):
    x = x_ref[:]
    h = lax.dot_general(x, w1_ref[:], (((1,), (1,)), ((), ())),
                        preferred_element_type=jnp.float32)
    h = jnp.maximum(h + b1_ref[:], 0.0)
    h = lax.dot_general(h, w2_ref[:], (((1,), (1,)), ((), ())),
                        preferred_element_type=jnp.float32)
    h = jnp.maximum(h + b2_ref[:], 0.0)
    h = lax.dot_general(h, w3_ref[:], (((1,), (1,)), ((), ())),
                        preferred_element_type=jnp.float32)
    o_ref[:] = h + b3_ref[:]


_mlp = pl.pallas_call(
    _mlp_body,
    out_shape=jax.ShapeDtypeStruct((B, 8), jnp.float32),
)


def kernel(user_features, user_ad_history, target_ad_id, emb_table, dense_table,
           fc1_w, fc1_b, fc2_w, fc2_b, fc3_w, fc3_b):
    hist = (user_ad_history.astype(jnp.int32) + 1).reshape(B * L)
    tgt = target_ad_id.astype(jnp.int32)

    us, ud, ts, td = _sc_bag_sums(hist, tgt, emb_table, dense_table)

    x = jnp.concatenate([ud[:, :DD], td, us, ts, user_features], axis=1)

    w3p = jnp.zeros((8, 80), jnp.float32).at[:2, :].set(fc3_w)
    b3p = jnp.zeros((8,), jnp.float32).at[:2].set(fc3_b)
    out = _mlp(x, fc1_w, fc1_b.reshape(1, -1), fc2_w, fc2_b.reshape(1, -1),
               w3p, b3p.reshape(1, -1))
    return out[:, :2]


# speedup vs baseline: 3.5532x; 1.6506x over previous
import functools

import jax
import jax.numpy as jnp
from jax import lax
from jax.experimental import pallas as pl
from jax.experimental.pallas import tpu as pltpu
from jax.experimental.pallas import tpu_sc as plsc

B = 1024
L = 200
EM = 160
DD = 8
NC = 2
NS = 16
NW = NC * NS
BPW = B // NW
NCH = 5
CH = L // NCH
EV = EM // 16

_mesh = plsc.VectorSubcoreMesh(core_axis_name="c", subcore_axis_name="s")


@functools.partial(
    pl.kernel,
    out_type=[
        jax.ShapeDtypeStruct((B, EM), jnp.float32),
        jax.ShapeDtypeStruct((B, 16), jnp.float32),
        jax.ShapeDtypeStruct((B, EM), jnp.float32),
        jax.ShapeDtypeStruct((B, DD), jnp.float32),
    ],
    mesh=_mesh,
    compiler_params=pltpu.CompilerParams(use_tc_tiling_on_sc=False,
                                         needs_layout_passes=False),
    scratch_types=[
        pltpu.VMEM((BPW * L,), jnp.int32),
        pltpu.VMEM((BPW,), jnp.int32),
        pltpu.VMEM((NCH, CH, EM), jnp.float32),
        pltpu.VMEM((NCH, CH, DD), jnp.float32),
        pltpu.VMEM((BPW, EM), jnp.float32),
        pltpu.VMEM((BPW, 16), jnp.float32),
        pltpu.VMEM((BPW, EM), jnp.float32),
        pltpu.VMEM((BPW, DD), jnp.float32),
        pltpu.VMEM((32,), jnp.float32),
        pltpu.SemaphoreType.DMA,
    ],
)
def _sc_bag_sums(hist_hbm, tgt_hbm, emb_hbm, dense_hbm,
                 us_hbm, ud_hbm, ts_hbm, td_hbm,
                 idx_v, tgt_v, erows_v, drows_v, us_v, ud_v, trow_v, tdrow_v,
                 mask_v, sem):
    wid = lax.axis_index("s") * NC + lax.axis_index("c")
    base = wid * BPW

    pltpu.sync_copy(hist_hbm.at[pl.ds(base * L, BPW * L)], idx_v)
    pltpu.sync_copy(tgt_hbm.at[pl.ds(base, BPW)], tgt_v)

    pltpu.async_copy(emb_hbm.at[tgt_v], trow_v, sem).wait()
    pltpu.async_copy(dense_hbm.at[tgt_v], tdrow_v, sem).wait()

    lanes = lax.iota(jnp.int32, 16)
    row_off = jnp.where(lanes >= 8, 1, 0)
    col_idx = lanes & 7

    def bag_body(b, _):
        cps = []
        for c in range(NCH):
            sl = idx_v.at[pl.ds(b * L + c * CH, CH)]
            cps.append(pltpu.async_copy(emb_hbm.at[sl], erows_v.at[c], sem))
            cps.append(pltpu.async_copy(dense_hbm.at[sl], drows_v.at[c], sem))
        for cp in cps:
            cp.wait()

        accs = tuple(jnp.zeros((16,), jnp.float32) for _ in range(EV))
        for c in range(NCH):
            def erow_body(r, a, c=c):
                return tuple(x + erows_v[c, r, pl.ds(d * 16, 16)]
                             for d, x in enumerate(a))
            accs = lax.fori_loop(0, CH, erow_body, accs)
        for d, a in enumerate(accs):
            us_v[b, pl.ds(d * 16, 16)] = a

        dacc = jnp.zeros((16,), jnp.float32)
        for c in range(NCH):
            cidx = jnp.zeros((16,), jnp.int32) + c

            def drow_body(i, acc, cidx=cidx):
                ridx = 2 * i + row_off
                return acc + plsc.load_gather(drows_v, [cidx, ridx, col_idx])
            dacc = lax.fori_loop(0, CH // 2, drow_body, dacc)
        mask_v[pl.ds(0, 16)] = dacc
        hi = plsc.load_gather(mask_v, [lanes ^ 8])
        ud_v[b, :] = dacc + hi
        return 0

    lax.fori_loop(0, BPW, bag_body, 0)

    for g in range(BPW // 16):
        tv = tgt_v[pl.ds(g * 16, 16)]
        mask_v[pl.ds(g * 16, 16)] = jnp.where(tv != 0, 1.0, 0.0)

    def mask_body(j, _):
        bc = plsc.load_gather(mask_v, [jnp.zeros((16,), jnp.int32) + j])
        for d in range(EV):
            trow_v[j, pl.ds(d * 16, 16)] = trow_v[j, pl.ds(d * 16, 16)] * bc
        return 0

    lax.fori_loop(0, BPW, mask_body, 0)

    pltpu.sync_copy(us_v, us_hbm.at[pl.ds(base, BPW), :])
    pltpu.sync_copy(ud_v, ud_hbm.at[pl.ds(base, BPW), :])
    pltpu.sync_copy(trow_v, ts_hbm.at[pl.ds(base, BPW), :])
    pltpu.sync_copy(tdrow_v, td_hbm.at[pl.ds(base, BPW), :])


def _tr_body(src_ref, dst_ref):
    dst_ref[:] = src_ref[:].T


_TBLK = 2048
_NBLK = (100001 + _TBLK - 1) // _TBLK

_emb_tr = pl.pallas_call(
    _tr_body,
    grid=(_NBLK,),
    in_specs=[pl.BlockSpec((EM, _TBLK), lambda i: (0, i))],
    out_specs=pl.BlockSpec((_TBLK, EM), lambda i: (i, 0)),
    out_shape=jax.ShapeDtypeStruct((100001, EM), jnp.float32),
)

_dense_tr = pl.pallas_call(
    _tr_body,
    grid=(_NBLK,),
    in_specs=[pl.BlockSpec((DD, _TBLK), lambda i: (0, i))],
    out_specs=pl.BlockSpec((_TBLK, DD), lambda i: (i, 0)),
    out_shape=jax.ShapeDtypeStruct((100001, DD), jnp.float32),
)


def _mlp_body(x_ref, w1_ref, b1_ref, w2_ref, b2_ref, w3_ref, b3_ref, o_ref):
    x = x_ref[:]
    h = lax.dot_general(x, w1_ref[:], (((1,), (1,)), ((), ())),
                        preferred_element_type=jnp.float32)
    h = jnp.maximum(h + b1_ref[:], 0.0)
    h = lax.dot_general(h, w2_ref[:], (((1,), (1,)), ((), ())),
                        preferred_element_type=jnp.float32)
    h = jnp.maximum(h + b2_ref[:], 0.0)
    h = lax.dot_general(h, w3_ref[:], (((1,), (1,)), ((), ())),
                        preferred_element_type=jnp.float32)
    o_ref[:] = h + b3_ref[:]


_mlp = pl.pallas_call(
    _mlp_body,
    out_shape=jax.ShapeDtypeStruct((B, 8), jnp.float32),
)


def kernel(user_features, user_ad_history, target_ad_id, emb_table, dense_table,
           fc1_w, fc1_b, fc2_w, fc2_b, fc3_w, fc3_b):
    hist = (user_ad_history.astype(jnp.int32) + 1).reshape(B * L)
    tgt = target_ad_id.astype(jnp.int32)

    emb_rm = _emb_tr(jnp.transpose(emb_table))
    dense_rm = _dense_tr(jnp.transpose(dense_table))
    us, ud, ts, td = _sc_bag_sums(hist, tgt, emb_rm, dense_rm)

    x = jnp.concatenate([ud[:, :DD], td, us, ts, user_features], axis=1)

    w3p = jnp.zeros((8, 80), jnp.float32).at[:2, :].set(fc3_w)
    b3p = jnp.zeros((8,), jnp.float32).at[:2].set(fc3_b)
    out = _mlp(x, fc1_w, fc1_b.reshape(1, -1), fc2_w, fc2_b.reshape(1, -1),
               w3p, b3p.reshape(1, -1))
    return out[:, :2]
